# Initial kernel scaffold; baseline (speedup 1.0000x reference)
import jax
import jax.numpy as jnp
from jax.experimental import pallas as pl


def kernel(species, moves, ability, status1, holdItem, hp, lvl, att, defn, spe, spA, spD, pp, exp, species_emb, move_emb, ability_emb, status_emb, item_emb, W1, b1, W2, b2):
    raise NotImplementedError("write your pallas kernel here")


# speedup vs baseline: 5.0801x; 5.0801x over previous
import functools

import jax
import jax.numpy as jnp
from jax import lax
from jax.experimental import pallas as pl
from jax.experimental.pallas import tpu as pltpu
from jax.experimental.pallas import tpu_sc as plsc

NC, NS, L = 2, 16, 16
NW = NC * NS
IDX_CHUNK = 128


def _sc_gather(table, idx2, B):
    n_words = table.shape[0] * L
    flat_per_w = idx2.shape[1]
    out_words = flat_per_w * L
    n_steps = flat_per_w // L

    mesh = plsc.VectorSubcoreMesh(core_axis_name="c", subcore_axis_name="s")

    @functools.partial(
        pl.kernel,
        mesh=mesh,
        out_type=jax.ShapeDtypeStruct((B * 8 * L,), jnp.float32),
        scratch_types=[
            pltpu.VMEM((n_words,), jnp.float32),
            pltpu.VMEM((flat_per_w,), jnp.int32),
            pltpu.VMEM((out_words,), jnp.float32),
        ],
        compiler_params=pltpu.CompilerParams(needs_layout_passes=False),
    )
    def body(table_hbm, idx_hbm, out_hbm, table_v, idx_v, rows_v):
        wid = lax.axis_index("s") * NC + lax.axis_index("c")
        pltpu.sync_copy(table_hbm, table_v)
        pltpu.sync_copy(idx_hbm.at[wid], idx_v)

        viota = jax.lax.iota(jnp.int32, L)

        def step(k, c):
            flat = viota + k * L
            sbase = ((flat >> 3) << 7) + ((flat & 7) << 4)
            rows = idx_v[pl.ds(k * L, L)]
            gbase = rows << 4
            for col in range(L):
                vals = plsc.load_gather(table_v, [gbase + col])
                plsc.store_scatter(rows_v, [sbase + col], vals)
            return c

        lax.fori_loop(0, n_steps, step, 0)

        pltpu.sync_copy(rows_v, out_hbm.at[pl.ds(wid * out_words, out_words)])

    return body(table.reshape(-1), idx2)


def _mlp_body(g_ref, n_ref, w1a_ref, w1b_ref, b1_ref, w2_ref, b2_ref, o_ref):
    h = jnp.dot(g_ref[...], w1a_ref[...], preferred_element_type=jnp.float32)
    h = h + jnp.dot(n_ref[...], w1b_ref[...], preferred_element_type=jnp.float32)
    h = jnp.maximum(h + b1_ref[...], 0.0)
    o_ref[...] = jnp.dot(h, w2_ref[...], preferred_element_type=jnp.float32) + b2_ref[...]


def kernel(species, moves, ability, status1, holdItem, hp, lvl, att, defn, spe,
           spA, spD, pp, exp, species_emb, move_emb, ability_emb, status_emb,
           item_emb, W1, b1, W2, b2):
    B = species.shape[0]
    f32, i32 = jnp.float32, jnp.int32

    o_m = species_emb.shape[0]
    o_a = o_m + move_emb.shape[0]
    o_st = o_a + ability_emb.shape[0]
    o_it = o_st + status_emb.shape[0]
    table = jnp.concatenate(
        [species_emb, move_emb, ability_emb, status_emb, item_emb], axis=0)
    n_tab = table.shape[0]
    pad = (-n_tab) % 8
    if pad:
        table = jnp.concatenate([table, jnp.zeros((pad, L), f32)], axis=0)

    idx = jnp.stack([
        species.astype(i32),
        moves[:, 0].astype(i32) + o_m,
        moves[:, 1].astype(i32) + o_m,
        moves[:, 2].astype(i32) + o_m,
        moves[:, 3].astype(i32) + o_m,
        ability.astype(i32) + o_a,
        status1.astype(i32) + o_st,
        holdItem.astype(i32) + o_it,
    ], axis=1)
    idx2 = idx.reshape(NW, (B * 8) // NW)

    g = _sc_gather(table, idx2, B).reshape(B, 8 * L)

    n_raw = jnp.concatenate([
        hp[:, None].astype(f32), lvl[:, None].astype(f32),
        att[:, None].astype(f32), defn[:, None].astype(f32),
        spe[:, None].astype(f32), spA[:, None].astype(f32),
        spD[:, None].astype(f32), pp.astype(f32), exp[:, None].astype(f32),
        jnp.zeros((B, 4), f32),
    ], axis=1)
    w1a = W1[:128]
    w1b = jnp.concatenate([
        W1[128:129], W1[129:130] / 100.0, W1[130:135],
        jnp.broadcast_to(W1[135:136] / 4.0, (4, 128)), W1[136:137],
        jnp.zeros((4, 128), f32),
    ], axis=0)

    BLK = 1024
    out = pl.pallas_call(
        _mlp_body,
        grid=(B // BLK,),
        in_specs=[
            pl.BlockSpec((BLK, 128), lambda i: (i, 0)),
            pl.BlockSpec((BLK, 16), lambda i: (i, 0)),
            pl.BlockSpec((128, 128), lambda i: (0, 0)),
            pl.BlockSpec((16, 128), lambda i: (0, 0)),
            pl.BlockSpec((1, 128), lambda i: (0, 0)),
            pl.BlockSpec((128, 128), lambda i: (0, 0)),
            pl.BlockSpec((1, 128), lambda i: (0, 0)),
        ],
        out_specs=pl.BlockSpec((BLK, 128), lambda i: (i, 0)),
        out_shape=jax.ShapeDtypeStruct((B, 128), f32),
    )(g, n_raw, w1a, w1b, b1.reshape(1, 128), W2, b2.reshape(1, 128))
    return out


# speedup vs baseline: 6.3866x; 1.2572x over previous
import functools

import jax
import jax.numpy as jnp
from jax import lax
from jax.experimental import pallas as pl
from jax.experimental.pallas import tpu as pltpu
from jax.experimental.pallas import tpu_sc as plsc

NC, NS, L = 2, 16, 16
NW = NC * NS
IDX_CHUNK = 128


def _sc_gather(table, idx2, B):
    n_words = table.shape[0] * L
    flat_per_w = idx2.shape[1]
    out_words = flat_per_w * L
    n_steps = flat_per_w // L

    mesh = plsc.VectorSubcoreMesh(core_axis_name="c", subcore_axis_name="s")

    @functools.partial(
        pl.kernel,
        mesh=mesh,
        out_type=jax.ShapeDtypeStruct((B * 8 * L,), jnp.float32),
        scratch_types=[
            pltpu.VMEM((n_words,), jnp.float32),
            pltpu.VMEM((flat_per_w,), jnp.int32),
            pltpu.VMEM((out_words,), jnp.float32),
        ],
        compiler_params=pltpu.CompilerParams(needs_layout_passes=False),
    )
    def body(table_hbm, idx_hbm, out_hbm, table_v, idx_v, rows_v):
        wid = lax.axis_index("s") * NC + lax.axis_index("c")
        pltpu.sync_copy(table_hbm, table_v)
        pltpu.sync_copy(idx_hbm.at[wid], idx_v)

        viota = jax.lax.iota(jnp.int32, L)

        @plsc.parallel_loop(0, n_steps, unroll=4)
        def _step(k):
            flat = viota + k * L
            sbase = ((flat >> 3) << 7) + ((flat & 7) << 4)
            rows = idx_v[pl.ds(k * L, L)]
            gbase = rows << 4
            for col in range(L):
                vals = plsc.load_gather(table_v, [gbase + col])
                plsc.store_scatter(rows_v, [sbase + col], vals)

        pltpu.sync_copy(rows_v, out_hbm.at[pl.ds(wid * out_words, out_words)])

    return body(table.reshape(-1), idx2)


def _mlp_body(g_ref, n_ref, w1a_ref, w1b_ref, b1_ref, w2_ref, b2_ref, o_ref):
    h = jnp.dot(g_ref[...], w1a_ref[...], preferred_element_type=jnp.float32)
    h = h + jnp.dot(n_ref[...], w1b_ref[...], preferred_element_type=jnp.float32)
    h = jnp.maximum(h + b1_ref[...], 0.0)
    o_ref[...] = jnp.dot(h, w2_ref[...], preferred_element_type=jnp.float32) + b2_ref[...]


def kernel(species, moves, ability, status1, holdItem, hp, lvl, att, defn, spe,
           spA, spD, pp, exp, species_emb, move_emb, ability_emb, status_emb,
           item_emb, W1, b1, W2, b2):
    B = species.shape[0]
    f32, i32 = jnp.float32, jnp.int32

    o_m = species_emb.shape[0]
    o_a = o_m + move_emb.shape[0]
    o_st = o_a + ability_emb.shape[0]
    o_it = o_st + status_emb.shape[0]
    table = jnp.concatenate(
        [species_emb, move_emb, ability_emb, status_emb, item_emb], axis=0)
    n_tab = table.shape[0]
    pad = (-n_tab) % 8
    if pad:
        table = jnp.concatenate([table, jnp.zeros((pad, L), f32)], axis=0)

    idx = jnp.stack([
        species.astype(i32),
        moves[:, 0].astype(i32) + o_m,
        moves[:, 1].astype(i32) + o_m,
        moves[:, 2].astype(i32) + o_m,
        moves[:, 3].astype(i32) + o_m,
        ability.astype(i32) + o_a,
        status1.astype(i32) + o_st,
        holdItem.astype(i32) + o_it,
    ], axis=1)
    idx2 = idx.reshape(NW, (B * 8) // NW)

    g = _sc_gather(table, idx2, B).reshape(B, 8 * L)

    n_raw = jnp.concatenate([
        hp[:, None].astype(f32), lvl[:, None].astype(f32),
        att[:, None].astype(f32), defn[:, None].astype(f32),
        spe[:, None].astype(f32), spA[:, None].astype(f32),
        spD[:, None].astype(f32), pp.astype(f32), exp[:, None].astype(f32),
        jnp.zeros((B, 4), f32),
    ], axis=1)
    w1a = W1[:128]
    w1b = jnp.concatenate([
        W1[128:129], W1[129:130] / 100.0, W1[130:135],
        jnp.broadcast_to(W1[135:136] / 4.0, (4, 128)), W1[136:137],
        jnp.zeros((4, 128), f32),
    ], axis=0)

    BLK = 1024
    out = pl.pallas_call(
        _mlp_body,
        grid=(B // BLK,),
        in_specs=[
            pl.BlockSpec((BLK, 128), lambda i: (i, 0)),
            pl.BlockSpec((BLK, 16), lambda i: (i, 0)),
            pl.BlockSpec((128, 128), lambda i: (0, 0)),
            pl.BlockSpec((16, 128), lambda i: (0, 0)),
            pl.BlockSpec((1, 128), lambda i: (0, 0)),
            pl.BlockSpec((128, 128), lambda i: (0, 0)),
            pl.BlockSpec((1, 128), lambda i: (0, 0)),
        ],
        out_specs=pl.BlockSpec((BLK, 128), lambda i: (i, 0)),
        out_shape=jax.ShapeDtypeStruct((B, 128), f32),
    )(g, n_raw, w1a, w1b, b1.reshape(1, 128), W2, b2.reshape(1, 128))
    return out
